# Initial kernel scaffold; baseline (speedup 1.0000x reference)
import jax
import jax.numpy as jnp
from jax.experimental import pallas as pl


def kernel(feats, vel_xy, coords):
    raise NotImplementedError("write your pallas kernel here")


# speedup vs baseline: 74.1247x; 74.1247x over previous
import jax
import jax.numpy as jnp
from jax.experimental import pallas as pl

_ROWS = 200000
_CH = 64
_BLOCK_ROWS = 2000


def _imap(i):
    return i, jnp.int32(0)


def _gate_mix_kernel(feats_ref, vel_ref, out_ref):
    f = feats_ref[...]
    v = vel_ref[...]
    speed = jnp.abs(v[:, 0]) + jnp.abs(v[:, 1])
    s = jnp.sum(jnp.abs(f), axis=1)
    diff = s / jnp.maximum(s, 1e-6)
    gate = jnp.exp(-diff) / (1.0 + 0.25 * speed)
    out_ref[...] = (1.0 - gate)[:, None] * f


def kernel(feats, vel_xy, coords):
    del coords
    n, c = feats.shape
    feats = feats.astype(jnp.float32)
    vel_xy = vel_xy.astype(jnp.float32)
    br = _BLOCK_ROWS if n == _ROWS else n
    grid = (n // br,)
    return pl.pallas_call(
        _gate_mix_kernel,
        grid=grid,
        in_specs=[
            pl.BlockSpec((br, c), _imap),
            pl.BlockSpec((br, 2), _imap),
        ],
        out_specs=pl.BlockSpec((br, c), _imap),
        out_shape=jax.ShapeDtypeStruct((n, c), jnp.float32),
    )(feats, vel_xy)


# speedup vs baseline: 85.7005x; 1.1562x over previous
import jax
import jax.numpy as jnp
from jax.experimental import pallas as pl

_ROWS = 200000
_CH = 64
_BLOCK_ROWS = 8000


def _imap(i):
    return i, jnp.int32(0)


def _gate_mix_kernel(feats_ref, vel_ref, out_ref):
    f = feats_ref[...]
    v = vel_ref[...]
    speed = jnp.abs(v[:, 0]) + jnp.abs(v[:, 1])
    s = jnp.sum(jnp.abs(f), axis=1)
    diff = s / jnp.maximum(s, 1e-6)
    gate = jnp.exp(-diff) / (1.0 + 0.25 * speed)
    out_ref[...] = (1.0 - gate)[:, None] * f


def kernel(feats, vel_xy, coords):
    del coords
    n, c = feats.shape
    feats = feats.astype(jnp.float32)
    vel_xy = vel_xy.astype(jnp.float32)
    br = _BLOCK_ROWS if n == _ROWS else n
    grid = (n // br,)
    return pl.pallas_call(
        _gate_mix_kernel,
        grid=grid,
        in_specs=[
            pl.BlockSpec((br, c), _imap),
            pl.BlockSpec((br, 2), _imap),
        ],
        out_specs=pl.BlockSpec((br, c), _imap),
        out_shape=jax.ShapeDtypeStruct((n, c), jnp.float32),
    )(feats, vel_xy)
